# baseline (speedup 1.0000x reference)
import functools

import jax
import jax.numpy as jnp
from jax import lax
from jax.experimental import pallas as pl
from jax.experimental.pallas import tpu as pltpu
from jax.experimental.pallas import tpu_sc as plsc

H = 1024
I = 512
E = 8
NG = 4
T = 2048
BLK = 256
LOG_BLK = 8
NBLK = 24
NSLOT = NBLK * BLK
NTILES = 32


def _routing_kernel(x_ref, gw_ref, eb_ref, slot_ref, w2_ref, s2_ref, be_ref):
    x = x_ref[...]
    gw = gw_ref[...]
    logits_t = lax.dot_general(gw, x, (((1,), (1,)), ((), ())),
                               preferred_element_type=jnp.float32)
    scores = jax.nn.sigmoid(logits_t)
    sfc = scores + eb_ref[...]
    s = [sfc[e:e + 1, :] for e in range(E)]
    sc = [scores[e:e + 1, :] for e in range(E)]
    g = [s[2 * i] + s[2 * i + 1] for i in range(NG)]
    gsel = []
    for i in range(NG):
        r = jnp.zeros_like(g[0])
        for j in range(NG):
            if j == i:
                continue
            beats = (g[j] > g[i]) | ((g[j] == g[i]) & (j < i))
            r = r + beats.astype(jnp.float32)
        gsel.append(r < 2.0)
    tmp = [jnp.where(gsel[e // 2], s[e], 0.0) for e in range(E)]
    esel = []
    for e in range(E):
        r = jnp.zeros_like(g[0])
        for e2 in range(E):
            if e2 == e:
                continue
            beats = (tmp[e2] > tmp[e]) | ((tmp[e2] == tmp[e]) & (e2 < e))
            r = r + beats.astype(jnp.float32)
        esel.append(r < 2.0)
    w = [jnp.where(esel[e], sc[e], 0.0) for e in range(E)]
    wsum = w[0]
    for e in range(1, E):
        wsum = wsum + w[e]
    inv = 1.0 / (wsum + 1e-20)
    wn = [w[e] * inv for e in range(E)]
    sel_f = [esel[e].astype(jnp.float32) for e in range(E)]
    sel_mat = jnp.concatenate(sel_f, axis=0)

    iot_r = lax.broadcasted_iota(jnp.int32, (T, T), 0)
    iot_c = lax.broadcasted_iota(jnp.int32, (T, T), 1)
    tril = (iot_r < iot_c).astype(jnp.float32)
    rank_f = lax.dot_general(sel_mat, tril, (((1,), (0,)), ((), ())),
                             preferred_element_type=jnp.float32)
    counts = lax.dot_general(sel_mat, jnp.ones((T, 1), jnp.float32),
                             (((1,), (0,)), ((), ())),
                             preferred_element_type=jnp.float32)
    c_i = counts.astype(jnp.int32)
    pc = ((c_i + BLK - 1) >> LOG_BLK) << LOG_BLK
    e_r = lax.broadcasted_iota(jnp.int32, (E, E), 0)
    e_c = lax.broadcasted_iota(jnp.int32, (E, E), 1)
    tril8 = (e_c < e_r).astype(jnp.float32)
    off = lax.dot_general(tril8, pc.astype(jnp.float32),
                          (((1,), (0,)), ((), ())),
                          preferred_element_type=jnp.float32
                          ).astype(jnp.int32)
    rank_i = rank_f.astype(jnp.int32)
    slot_mat = jnp.where(sel_mat > 0.5, off + rank_i, -1)
    slot_ref[...] = slot_mat

    slot_f = (off + rank_i).astype(jnp.float32)
    cb = jnp.zeros((1, T), jnp.float32)
    slotA = jnp.zeros((1, T), jnp.float32)
    slotB = jnp.zeros((1, T), jnp.float32)
    wA = jnp.zeros((1, T), jnp.float32)
    wB = jnp.zeros((1, T), jnp.float32)
    for e in range(E):
        f = sel_f[e]
        first = f * (cb == 0.0).astype(jnp.float32)
        second = f * (cb == 1.0).astype(jnp.float32)
        row = slot_f[e:e + 1, :]
        slotA = slotA + first * row
        slotB = slotB + second * row
        wA = wA + first * wn[e]
        wB = wB + second * wn[e]
        cb = cb + f
    s2_ref[...] = jnp.concatenate([slotA, slotB], axis=0).astype(jnp.int32)
    zero = jnp.zeros((1, T), jnp.float32)
    w2_t = jnp.concatenate([wA, wB] + [zero] * (E - 2), axis=0)
    w2_ref[...] = w2_t.T

    bidx = lax.broadcasted_iota(jnp.int32, (E, 32), 1)
    boff = off >> LOG_BLK
    ge = (bidx >= boff).astype(jnp.int32)
    cnt = jnp.sum(ge, axis=0, keepdims=True)
    be_row = jnp.clip(cnt - 1, 0, E - 1)
    nb = jnp.sum(pc, axis=0, keepdims=True) >> LOG_BLK
    bi1 = lax.broadcasted_iota(jnp.int32, (1, 32), 1)
    be_row = jnp.where(bi1 == 31, nb, be_row)
    be_ref[...] = be_row + jnp.zeros((E, 32), jnp.int32)


def _build_s1():
    mesh = plsc.VectorSubcoreMesh(core_axis_name="c", subcore_axis_name="s")
    per = (E * T) // NTILES
    nch = per // 128

    @functools.partial(
        pl.kernel,
        mesh=mesh,
        out_type=jax.ShapeDtypeStruct((NSLOT + NTILES,), jnp.int32),
        scratch_types=[
            pltpu.VMEM((nch, 128), jnp.int32),
            pltpu.VMEM((nch, 128), jnp.int32),
            pltpu.SemaphoreType.DMA,
        ],
    )
    def _s1(slot_hbm, stt_hbm, idx_v, val_v, sem):
        c = lax.axis_index("c")
        s = lax.axis_index("s")
        wid = s * 2 + c
        base = wid * per
        pltpu.sync_copy(slot_hbm.at[pl.ds(wid * nch, nch)], idx_v)
        dump = NSLOT + wid
        for j in range(nch):
            for k in range(128 // 16):
                v = idx_v[j, pl.ds(k * 16, 16)]
                flat = base + j * 128 + k * 16 + lax.iota(jnp.int32, 16)
                val_v[j, pl.ds(k * 16, 16)] = flat & (T - 1)
                idx_v[j, pl.ds(k * 16, 16)] = jnp.where(v >= 0, v, dump)
        copies = []
        for j in range(nch):
            copies.append(pltpu.async_copy(
                val_v.at[j], stt_hbm.at[idx_v.at[j]], sem))
        for cp in copies:
            cp.wait()

    return _s1


def _build_gather(n_rows, n_cols, table_rows, clamp):
    mesh = plsc.VectorSubcoreMesh(core_axis_name="c", subcore_axis_name="s")
    rows_per = n_rows // NTILES
    chunk = 64

    @functools.partial(
        pl.kernel,
        mesh=mesh,
        out_type=jax.ShapeDtypeStruct((n_rows, n_cols), jnp.int32),
        scratch_types=[
            pltpu.VMEM((rows_per,), jnp.int32),
            pltpu.VMEM((rows_per, n_cols), jnp.int32),
            pltpu.SemaphoreType.DMA,
        ],
    )
    def _gather(idx_hbm, tab_hbm, out_hbm, idx_v, rows_v, sem):
        c = lax.axis_index("c")
        s = lax.axis_index("s")
        wid = s * 2 + c
        base = wid * rows_per
        pltpu.sync_copy(idx_hbm.at[pl.ds(base, rows_per)], idx_v)
        if clamp:
            def body(j, carry):
                v = idx_v[pl.ds(j * 16, 16)]
                idx_v[pl.ds(j * 16, 16)] = jnp.minimum(
                    jnp.maximum(v, 0), table_rows - 1)
                return carry

            lax.fori_loop(0, rows_per // 16, body, 0)
        copies = []
        for k in range(rows_per // chunk):
            copies.append(pltpu.async_copy(
                tab_hbm.at[idx_v.at[pl.ds(k * chunk, chunk)]],
                rows_v.at[pl.ds(k * chunk, chunk)], sem))
        for cp in copies:
            cp.wait()
        pltpu.sync_copy(rows_v, out_hbm.at[pl.ds(base, rows_per)])

    return _gather


@functools.lru_cache(maxsize=None)
def _sc_kernels():
    return (_build_s1(),
            _build_gather(NSLOT, H // 2, T, clamp=True),
            _build_gather(2 * T, H // 2, NSLOT, clamp=False))


def _s1_scatter(slot_flat):
    return _sc_kernels()[0](slot_flat)


def _s2_gather(stt, x_bits):
    return _sc_kernels()[1](stt, x_bits)


def _s3_gather(s2_flat, y_bits):
    return _sc_kernels()[2](s2_flat, y_bits)


def _ffn_sparse_kernel(be_ref, disp_ref, wg_ref, wu_ref, wd_ref, y_ref):
    b = pl.program_id(0)
    nb = be_ref[31]

    @pl.when(b < nb)
    def _():
        x = disp_ref[...]
        hg = lax.dot_general(x, wg_ref[0], (((1,), (1,)), ((), ())),
                             preferred_element_type=jnp.float32)
        hu = lax.dot_general(x, wu_ref[0], (((1,), (1,)), ((), ())),
                             preferred_element_type=jnp.float32)
        inter = (hg * jax.nn.sigmoid(hg)) * hu
        y = lax.dot_general(inter.astype(jnp.bfloat16), wd_ref[0],
                            (((1,), (1,)), ((), ())),
                            preferred_element_type=jnp.float32)
        y_ref[...] = y.astype(jnp.bfloat16)


def _final_kernel(x_ref, wsg_ref, wsu_ref, wsd_ref, y2_ref, w2_ref, out_ref):
    x = x_ref[...]
    hg = lax.dot_general(x, wsg_ref[...], (((1,), (1,)), ((), ())),
                         preferred_element_type=jnp.float32)
    hu = lax.dot_general(x, wsu_ref[...], (((1,), (1,)), ((), ())),
                         preferred_element_type=jnp.float32)
    inter = (hg * jax.nn.sigmoid(hg)) * hu
    sh = lax.dot_general(inter.astype(jnp.bfloat16), wsd_ref[...],
                         (((1,), (1,)), ((), ())),
                         preferred_element_type=jnp.float32)
    yA = y2_ref[0].astype(jnp.float32)
    yB = y2_ref[1].astype(jnp.float32)
    wA = w2_ref[:, 0:1]
    wB = w2_ref[:, 1:2]
    out_ref[...] = sh + yA * wA + yB * wB


def kernel(hidden_states, gate_w, e_bias, gate_proj, up_proj, down_proj,
           shared_gate_w, shared_up_w, shared_down_w):
    b, ss, h = hidden_states.shape
    x = hidden_states.reshape(T, H)

    slot_mat, w2, s2, be_out = pl.pallas_call(
        _routing_kernel,
        out_shape=(
            jax.ShapeDtypeStruct((E, T), jnp.int32),
            jax.ShapeDtypeStruct((T, E), jnp.float32),
            jax.ShapeDtypeStruct((2, T), jnp.int32),
            jax.ShapeDtypeStruct((E, 32), jnp.int32),
        ),
    )(x, gate_w, e_bias.reshape(E, 1))

    stt = _s1_scatter(slot_mat.reshape(E * T // 128, 128))

    bf16 = jnp.bfloat16
    x_bf = x.astype(bf16)
    x_bits = lax.bitcast_convert_type(
        x_bf.reshape(T, H // 2, 2), jnp.int32)
    disp_bits = _s2_gather(stt, x_bits)
    disp_bf = lax.bitcast_convert_type(disp_bits, bf16).reshape(NSLOT, H)

    wg8 = gate_proj.astype(bf16)
    wu8 = up_proj.astype(bf16)
    wd8 = down_proj.astype(bf16)
    be_arr = be_out[0]

    grid_spec = pltpu.PrefetchScalarGridSpec(
        num_scalar_prefetch=1,
        grid=(NBLK,),
        in_specs=[
            pl.BlockSpec((BLK, H), lambda bb, be: (bb, 0)),
            pl.BlockSpec((1, I, H), lambda bb, be: (be[bb], 0, 0)),
            pl.BlockSpec((1, I, H), lambda bb, be: (be[bb], 0, 0)),
            pl.BlockSpec((1, H, I), lambda bb, be: (be[bb], 0, 0)),
        ],
        out_specs=pl.BlockSpec((BLK, H), lambda bb, be: (bb, 0)),
    )
    y_bf = pl.pallas_call(
        _ffn_sparse_kernel,
        grid_spec=grid_spec,
        out_shape=jax.ShapeDtypeStruct((NSLOT, H), bf16),
    )(be_arr, disp_bf, wg8, wu8, wd8)

    y_bits = lax.bitcast_convert_type(
        y_bf.reshape(NSLOT, H // 2, 2), jnp.int32)
    y2_bits = _s3_gather(s2.reshape(2 * T), y_bits)
    y2 = lax.bitcast_convert_type(y2_bits, bf16).reshape(2, T, H)

    out = pl.pallas_call(
        _final_kernel,
        out_shape=jax.ShapeDtypeStruct((T, H), jnp.float32),
    )(x_bf, shared_gate_w.astype(bf16), shared_up_w.astype(bf16),
      shared_down_w.astype(bf16), y2, w2)

    return out.reshape(b, ss, h)


# speedup vs baseline: 4.4104x; 4.4104x over previous
import functools

import jax
import jax.numpy as jnp
from jax import lax
from jax.experimental import pallas as pl
from jax.experimental.pallas import tpu as pltpu
from jax.experimental.pallas import tpu_sc as plsc

H = 1024
I = 512
E = 8
NG = 4
T = 2048
BLK = 256
LOG_BLK = 8
NBLK = 24
NSLOT = NBLK * BLK
NTILES = 32


def _routing_kernel(x_ref, gw_ref, eb_ref, w2_ref, s2_ref, be_ref):
    x = x_ref[...]
    gw = gw_ref[...]
    logits_t = lax.dot_general(gw, x, (((1,), (1,)), ((), ())),
                               preferred_element_type=jnp.float32)
    scores = jax.nn.sigmoid(logits_t)
    sfc = scores + eb_ref[...]
    s = [sfc[e:e + 1, :] for e in range(E)]
    sc = [scores[e:e + 1, :] for e in range(E)]
    g = [s[2 * i] + s[2 * i + 1] for i in range(NG)]
    gsel = []
    for i in range(NG):
        r = jnp.zeros_like(g[0])
        for j in range(NG):
            if j == i:
                continue
            beats = (g[j] > g[i]) | ((g[j] == g[i]) & (j < i))
            r = r + beats.astype(jnp.float32)
        gsel.append(r < 2.0)
    tmp = [jnp.where(gsel[e // 2], s[e], 0.0) for e in range(E)]
    esel = []
    for e in range(E):
        r = jnp.zeros_like(g[0])
        for e2 in range(E):
            if e2 == e:
                continue
            beats = (tmp[e2] > tmp[e]) | ((tmp[e2] == tmp[e]) & (e2 < e))
            r = r + beats.astype(jnp.float32)
        esel.append(r < 2.0)
    w = [jnp.where(esel[e], sc[e], 0.0) for e in range(E)]
    wsum = w[0]
    for e in range(1, E):
        wsum = wsum + w[e]
    inv = 1.0 / (wsum + 1e-20)
    wn = [w[e] * inv for e in range(E)]
    sel_f = [esel[e].astype(jnp.float32) for e in range(E)]
    sel_mat = jnp.concatenate(sel_f, axis=0)

    iot_r = lax.broadcasted_iota(jnp.int32, (T, T), 0)
    iot_c = lax.broadcasted_iota(jnp.int32, (T, T), 1)
    tril = (iot_r < iot_c).astype(jnp.float32)
    rank_f = lax.dot_general(sel_mat, tril, (((1,), (0,)), ((), ())),
                             preferred_element_type=jnp.float32)
    counts = lax.dot_general(sel_mat, jnp.ones((T, 1), jnp.float32),
                             (((1,), (0,)), ((), ())),
                             preferred_element_type=jnp.float32)
    c_i = counts.astype(jnp.int32)
    pc = ((c_i + BLK - 1) >> LOG_BLK) << LOG_BLK
    e_r = lax.broadcasted_iota(jnp.int32, (E, E), 0)
    e_c = lax.broadcasted_iota(jnp.int32, (E, E), 1)
    tril8 = (e_c < e_r).astype(jnp.float32)
    off = lax.dot_general(tril8, pc.astype(jnp.float32),
                          (((1,), (0,)), ((), ())),
                          preferred_element_type=jnp.float32
                          ).astype(jnp.int32)
    rank_i = rank_f.astype(jnp.int32)

    slot_f = (off + rank_i).astype(jnp.float32)
    cb = jnp.zeros((1, T), jnp.float32)
    slotA = jnp.zeros((1, T), jnp.float32)
    slotB = jnp.zeros((1, T), jnp.float32)
    wA = jnp.zeros((1, T), jnp.float32)
    wB = jnp.zeros((1, T), jnp.float32)
    for e in range(E):
        f = sel_f[e]
        first = f * (cb == 0.0).astype(jnp.float32)
        second = f * (cb == 1.0).astype(jnp.float32)
        row = slot_f[e:e + 1, :]
        slotA = slotA + first * row
        slotB = slotB + second * row
        wA = wA + first * wn[e]
        wB = wB + second * wn[e]
        cb = cb + f
    s2_ref[...] = jnp.concatenate([slotA, slotB], axis=0).astype(jnp.int32)
    zero = jnp.zeros((1, T), jnp.float32)
    w2_t = jnp.concatenate([wA, wB] + [zero] * (E - 2), axis=0)
    w2_ref[...] = w2_t.T

    bidx = lax.broadcasted_iota(jnp.int32, (E, 32), 1)
    boff = off >> LOG_BLK
    ge = (bidx >= boff).astype(jnp.int32)
    cnt = jnp.sum(ge, axis=0, keepdims=True)
    be_row = jnp.clip(cnt - 1, 0, E - 1)
    nb = jnp.sum(pc, axis=0, keepdims=True) >> LOG_BLK
    bi1 = lax.broadcasted_iota(jnp.int32, (1, 32), 1)
    be_row = jnp.where(bi1 == 31, nb, be_row)
    be_ref[...] = be_row + jnp.zeros((E, 32), jnp.int32)


def _build_dispatch():
    mesh = plsc.VectorSubcoreMesh(core_axis_name="c", subcore_axis_name="s")
    rows_per = (2 * T) // NTILES

    @functools.partial(
        pl.kernel,
        mesh=mesh,
        out_type=jax.ShapeDtypeStruct((NSLOT, H // 2), jnp.int32),
        scratch_types=[
            pltpu.VMEM((rows_per,), jnp.int32),
            pltpu.VMEM((rows_per, H // 2), jnp.int32),
            pltpu.SemaphoreType.DMA,
        ],
    )
    def _dispatch(s2_hbm, x_hbm, disp_hbm, idx_v, rows_v, sem):
        c = lax.axis_index("c")
        s = lax.axis_index("s")
        wid = s * 2 + c
        flat0 = pl.multiple_of(wid * rows_per, rows_per)
        t0 = pl.multiple_of(flat0 & (T - 1), rows_per)
        pltpu.sync_copy(x_hbm.at[pl.ds(t0, rows_per)], rows_v)
        pltpu.sync_copy(s2_hbm.at[pl.ds(flat0, rows_per)], idx_v)
        pltpu.async_copy(rows_v, disp_hbm.at[idx_v], sem).wait()

    return _dispatch


def _build_gather(n_rows, n_cols, table_rows):
    mesh = plsc.VectorSubcoreMesh(core_axis_name="c", subcore_axis_name="s")
    rows_per = n_rows // NTILES
    chunk = 64

    @functools.partial(
        pl.kernel,
        mesh=mesh,
        out_type=jax.ShapeDtypeStruct((n_rows, n_cols), jnp.int32),
        scratch_types=[
            pltpu.VMEM((rows_per,), jnp.int32),
            pltpu.VMEM((rows_per, n_cols), jnp.int32),
            pltpu.SemaphoreType.DMA,
        ],
    )
    def _gather(idx_hbm, tab_hbm, out_hbm, idx_v, rows_v, sem):
        c = lax.axis_index("c")
        s = lax.axis_index("s")
        wid = s * 2 + c
        base = pl.multiple_of(wid * rows_per, rows_per)
        pltpu.sync_copy(idx_hbm.at[pl.ds(base, rows_per)], idx_v)
        copies = []
        for k in range(rows_per // chunk):
            copies.append(pltpu.async_copy(
                tab_hbm.at[idx_v.at[pl.ds(k * chunk, chunk)]],
                rows_v.at[pl.ds(k * chunk, chunk)], sem))
        for cp in copies:
            cp.wait()
        pltpu.sync_copy(rows_v, out_hbm.at[pl.ds(base, rows_per)])

    return _gather


@functools.lru_cache(maxsize=None)
def _sc_kernels():
    return (_build_dispatch(),
            _build_gather(2 * T, H // 2, NSLOT))


def _sc_dispatch(s2_flat, x_bits):
    return _sc_kernels()[0](s2_flat, x_bits)


def _s3_gather(s2_flat, y_bits):
    return _sc_kernels()[1](s2_flat, y_bits)


def _ffn_sparse_kernel(be_ref, disp_ref, wg_ref, wu_ref, wd_ref, y_ref):
    b = pl.program_id(0)
    nb = be_ref[31]

    @pl.when(b < nb)
    def _():
        x = disp_ref[...]
        hg = lax.dot_general(x, wg_ref[0], (((1,), (1,)), ((), ())),
                             preferred_element_type=jnp.float32)
        hu = lax.dot_general(x, wu_ref[0], (((1,), (1,)), ((), ())),
                             preferred_element_type=jnp.float32)
        inter = (hg * jax.nn.sigmoid(hg)) * hu
        y = lax.dot_general(inter.astype(jnp.bfloat16), wd_ref[0],
                            (((1,), (1,)), ((), ())),
                            preferred_element_type=jnp.float32)
        y_ref[...] = y.astype(jnp.bfloat16)


def _final_kernel(x_ref, wsg_ref, wsu_ref, wsd_ref, y2_ref, w2_ref, out_ref):
    x = x_ref[...]
    hg = lax.dot_general(x, wsg_ref[...], (((1,), (1,)), ((), ())),
                         preferred_element_type=jnp.float32)
    hu = lax.dot_general(x, wsu_ref[...], (((1,), (1,)), ((), ())),
                         preferred_element_type=jnp.float32)
    inter = (hg * jax.nn.sigmoid(hg)) * hu
    sh = lax.dot_general(inter.astype(jnp.bfloat16), wsd_ref[...],
                         (((1,), (1,)), ((), ())),
                         preferred_element_type=jnp.float32)
    yA = y2_ref[0].astype(jnp.float32)
    yB = y2_ref[1].astype(jnp.float32)
    wA = w2_ref[:, 0:1]
    wB = w2_ref[:, 1:2]
    out_ref[...] = sh + yA * wA + yB * wB


def kernel(hidden_states, gate_w, e_bias, gate_proj, up_proj, down_proj,
           shared_gate_w, shared_up_w, shared_down_w):
    b, ss, h = hidden_states.shape
    x = hidden_states.reshape(T, H)

    w2, s2, be_out = pl.pallas_call(
        _routing_kernel,
        out_shape=(
            jax.ShapeDtypeStruct((T, E), jnp.float32),
            jax.ShapeDtypeStruct((2, T), jnp.int32),
            jax.ShapeDtypeStruct((E, 32), jnp.int32),
        ),
    )(x, gate_w, e_bias.reshape(E, 1))

    bf16 = jnp.bfloat16
    x_bf = x.astype(bf16)
    x_bits = lax.bitcast_convert_type(
        x_bf.reshape(T, H // 2, 2), jnp.int32)
    disp_bits = _sc_dispatch(s2.reshape(2 * T), x_bits)
    disp_bf = lax.bitcast_convert_type(disp_bits, bf16).reshape(NSLOT, H)

    wg8 = gate_proj.astype(bf16)
    wu8 = up_proj.astype(bf16)
    wd8 = down_proj.astype(bf16)
    be_arr = be_out[0]

    grid_spec = pltpu.PrefetchScalarGridSpec(
        num_scalar_prefetch=1,
        grid=(NBLK,),
        in_specs=[
            pl.BlockSpec((BLK, H), lambda bb, be: (bb, 0)),
            pl.BlockSpec((1, I, H), lambda bb, be: (be[bb], 0, 0)),
            pl.BlockSpec((1, I, H), lambda bb, be: (be[bb], 0, 0)),
            pl.BlockSpec((1, H, I), lambda bb, be: (be[bb], 0, 0)),
        ],
        out_specs=pl.BlockSpec((BLK, H), lambda bb, be: (bb, 0)),
    )
    y_bf = pl.pallas_call(
        _ffn_sparse_kernel,
        grid_spec=grid_spec,
        out_shape=jax.ShapeDtypeStruct((NSLOT, H), bf16),
    )(be_arr, disp_bf, wg8, wu8, wd8)

    y_bits = lax.bitcast_convert_type(
        y_bf.reshape(NSLOT, H // 2, 2), jnp.int32)
    y2_bits = _s3_gather(s2.reshape(2 * T), y_bits)
    y2 = lax.bitcast_convert_type(y2_bits, bf16).reshape(2, T, H)

    out = pl.pallas_call(
        _final_kernel,
        out_shape=jax.ShapeDtypeStruct((T, H), jnp.float32),
    )(x_bf, shared_gate_w.astype(bf16), shared_up_w.astype(bf16),
      shared_down_w.astype(bf16), y2, w2)

    return out.reshape(b, ss, h)


# speedup vs baseline: 16.7496x; 3.7978x over previous
import functools

import jax
import jax.numpy as jnp
from jax import lax
from jax.experimental import pallas as pl
from jax.experimental.pallas import tpu as pltpu
from jax.experimental.pallas import tpu_sc as plsc

H = 1024
I = 512
E = 8
NG = 4
T = 2048
BLK = 256
LOG_BLK = 8
NBLK = 24
NSLOT = NBLK * BLK
NTILES = 32


def _routing_kernel(x_ref, gw_ref, eb_ref, w2_ref, s2_ref, be_ref):
    x = x_ref[...]
    gw = gw_ref[...]
    logits_t = lax.dot_general(gw, x, (((1,), (1,)), ((), ())),
                               preferred_element_type=jnp.float32)
    scores = jax.nn.sigmoid(logits_t)
    sfc = scores + eb_ref[...]
    s = [sfc[e:e + 1, :] for e in range(E)]
    sc = [scores[e:e + 1, :] for e in range(E)]
    g = [s[2 * i] + s[2 * i + 1] for i in range(NG)]
    gsel = []
    for i in range(NG):
        r = jnp.zeros_like(g[0])
        for j in range(NG):
            if j == i:
                continue
            beats = (g[j] > g[i]) | ((g[j] == g[i]) & (j < i))
            r = r + beats.astype(jnp.float32)
        gsel.append(r < 2.0)
    tmp = [jnp.where(gsel[e // 2], s[e], 0.0) for e in range(E)]
    esel = []
    for e in range(E):
        r = jnp.zeros_like(g[0])
        for e2 in range(E):
            if e2 == e:
                continue
            beats = (tmp[e2] > tmp[e]) | ((tmp[e2] == tmp[e]) & (e2 < e))
            r = r + beats.astype(jnp.float32)
        esel.append(r < 2.0)
    w = [jnp.where(esel[e], sc[e], 0.0) for e in range(E)]
    wsum = w[0]
    for e in range(1, E):
        wsum = wsum + w[e]
    inv = 1.0 / (wsum + 1e-20)
    wn = [w[e] * inv for e in range(E)]
    sel_f = [esel[e].astype(jnp.float32) for e in range(E)]
    sel_mat = jnp.concatenate(sel_f, axis=0)

    iot_r = lax.broadcasted_iota(jnp.int32, (T, T), 0)
    iot_c = lax.broadcasted_iota(jnp.int32, (T, T), 1)
    tril = (iot_r < iot_c).astype(jnp.float32)
    rank_f = lax.dot_general(sel_mat, tril, (((1,), (0,)), ((), ())),
                             preferred_element_type=jnp.float32)
    counts = lax.dot_general(sel_mat, jnp.ones((T, 1), jnp.float32),
                             (((1,), (0,)), ((), ())),
                             preferred_element_type=jnp.float32)
    c_i = counts.astype(jnp.int32)
    pc = ((c_i + BLK - 1) >> LOG_BLK) << LOG_BLK
    e_r = lax.broadcasted_iota(jnp.int32, (E, E), 0)
    e_c = lax.broadcasted_iota(jnp.int32, (E, E), 1)
    tril8 = (e_c < e_r).astype(jnp.float32)
    off = lax.dot_general(tril8, pc.astype(jnp.float32),
                          (((1,), (0,)), ((), ())),
                          preferred_element_type=jnp.float32
                          ).astype(jnp.int32)
    rank_i = rank_f.astype(jnp.int32)

    slot_f = (off + rank_i).astype(jnp.float32)
    cb = jnp.zeros((1, T), jnp.float32)
    slotA = jnp.zeros((1, T), jnp.float32)
    slotB = jnp.zeros((1, T), jnp.float32)
    wA = jnp.zeros((1, T), jnp.float32)
    wB = jnp.zeros((1, T), jnp.float32)
    for e in range(E):
        f = sel_f[e]
        first = f * (cb == 0.0).astype(jnp.float32)
        second = f * (cb == 1.0).astype(jnp.float32)
        row = slot_f[e:e + 1, :]
        slotA = slotA + first * row
        slotB = slotB + second * row
        wA = wA + first * wn[e]
        wB = wB + second * wn[e]
        cb = cb + f
    s2_ref[...] = jnp.concatenate([slotA, slotB], axis=0).astype(jnp.int32)
    zero = jnp.zeros((1, T), jnp.float32)
    w2_t = jnp.concatenate([wA, wB] + [zero] * (E - 2), axis=0)
    w2_ref[...] = w2_t.T

    bidx = lax.broadcasted_iota(jnp.int32, (E, 32), 1)
    boff = off >> LOG_BLK
    ge = (bidx >= boff).astype(jnp.int32)
    cnt = jnp.sum(ge, axis=0, keepdims=True)
    be_row = jnp.clip(cnt - 1, 0, E - 1)
    nb = jnp.sum(pc, axis=0, keepdims=True) >> LOG_BLK
    bi1 = lax.broadcasted_iota(jnp.int32, (1, 32), 1)
    be_row = jnp.where(bi1 == 31, nb, be_row)
    be_ref[...] = be_row + jnp.zeros((E, 32), jnp.int32)


RCH = 64


def _build_dispatch():
    mesh = plsc.VectorSubcoreMesh(core_axis_name="c", subcore_axis_name="s")
    rows_per = (2 * T) // NTILES
    nch = rows_per // RCH

    @functools.partial(
        pl.kernel,
        mesh=mesh,
        out_type=jax.ShapeDtypeStruct((NSLOT, H), jnp.float32),
        scratch_types=[
            pltpu.VMEM((nch, RCH), jnp.int32),
            pltpu.VMEM((RCH, H), jnp.float32),
            pltpu.SemaphoreType.DMA,
        ],
    )
    def _dispatch(s2_hbm, x_hbm, disp_hbm, idx_v, rows_v, sem):
        c = lax.axis_index("c")
        s = lax.axis_index("s")
        wid = s * 2 + c
        flat0 = pl.multiple_of(wid * rows_per, rows_per)
        t0 = pl.multiple_of(flat0 & (T - 1), rows_per)
        pltpu.sync_copy(s2_hbm.at[pl.ds(wid * nch, nch)], idx_v)
        for j in range(nch):
            pltpu.sync_copy(x_hbm.at[pl.ds(t0 + j * RCH, RCH)], rows_v)
            pltpu.async_copy(rows_v, disp_hbm.at[idx_v.at[j]], sem).wait()

    return _dispatch


def _build_gather(n_rows, n_cols, table_rows):
    mesh = plsc.VectorSubcoreMesh(core_axis_name="c", subcore_axis_name="s")
    rows_per = n_rows // NTILES
    nch = rows_per // RCH

    @functools.partial(
        pl.kernel,
        mesh=mesh,
        out_type=jax.ShapeDtypeStruct((n_rows, n_cols), jnp.float32),
        scratch_types=[
            pltpu.VMEM((nch, RCH), jnp.int32),
            pltpu.VMEM((RCH, n_cols), jnp.float32),
            pltpu.SemaphoreType.DMA,
        ],
    )
    def _gather(idx_hbm, tab_hbm, out_hbm, idx_v, rows_v, sem):
        c = lax.axis_index("c")
        s = lax.axis_index("s")
        wid = s * 2 + c
        base = pl.multiple_of(wid * rows_per, rows_per)
        pltpu.sync_copy(idx_hbm.at[pl.ds(wid * nch, nch)], idx_v)
        for j in range(nch):
            pltpu.async_copy(tab_hbm.at[idx_v.at[j]], rows_v, sem).wait()
            pltpu.sync_copy(rows_v, out_hbm.at[pl.ds(base + j * RCH, RCH)])

    return _gather


@functools.lru_cache(maxsize=None)
def _sc_kernels():
    return (_build_dispatch(),
            _build_gather(2 * T, H, NSLOT))


def _sc_dispatch(s2_chunked, x):
    return _sc_kernels()[0](s2_chunked, x)


def _s3_gather(s2_chunked, y):
    return _sc_kernels()[1](s2_chunked, y)


def _ffn_sparse_kernel(be_ref, disp_ref, wg_ref, wu_ref, wd_ref, y_ref):
    b = pl.program_id(0)
    nb = be_ref[31]

    @pl.when(b < nb)
    def _():
        x = disp_ref[...].astype(jnp.bfloat16)
        hg = lax.dot_general(x, wg_ref[0], (((1,), (1,)), ((), ())),
                             preferred_element_type=jnp.float32)
        hu = lax.dot_general(x, wu_ref[0], (((1,), (1,)), ((), ())),
                             preferred_element_type=jnp.float32)
        inter = (hg * jax.nn.sigmoid(hg)) * hu
        y_ref[...] = lax.dot_general(inter.astype(jnp.bfloat16), wd_ref[0],
                                     (((1,), (1,)), ((), ())),
                                     preferred_element_type=jnp.float32)


def _final_kernel(x_ref, wsg_ref, wsu_ref, wsd_ref, y2_ref, w2_ref, out_ref):
    x = x_ref[...]
    hg = lax.dot_general(x, wsg_ref[...], (((1,), (1,)), ((), ())),
                         preferred_element_type=jnp.float32)
    hu = lax.dot_general(x, wsu_ref[...], (((1,), (1,)), ((), ())),
                         preferred_element_type=jnp.float32)
    inter = (hg * jax.nn.sigmoid(hg)) * hu
    sh = lax.dot_general(inter.astype(jnp.bfloat16), wsd_ref[...],
                         (((1,), (1,)), ((), ())),
                         preferred_element_type=jnp.float32)
    yA = y2_ref[0]
    yB = y2_ref[1]
    wA = w2_ref[:, 0:1]
    wB = w2_ref[:, 1:2]
    out_ref[...] = sh + yA * wA + yB * wB


def kernel(hidden_states, gate_w, e_bias, gate_proj, up_proj, down_proj,
           shared_gate_w, shared_up_w, shared_down_w):
    b, ss, h = hidden_states.shape
    x = hidden_states.reshape(T, H)

    w2, s2, be_out = pl.pallas_call(
        _routing_kernel,
        out_shape=(
            jax.ShapeDtypeStruct((T, E), jnp.float32),
            jax.ShapeDtypeStruct((2, T), jnp.int32),
            jax.ShapeDtypeStruct((E, 32), jnp.int32),
        ),
    )(x, gate_w, e_bias.reshape(E, 1))

    bf16 = jnp.bfloat16
    x_bf = x.astype(bf16)
    s2_chunked = s2.reshape(2 * T // RCH, RCH)
    disp = _sc_dispatch(s2_chunked, x)

    wg8 = gate_proj.astype(bf16)
    wu8 = up_proj.astype(bf16)
    wd8 = down_proj.astype(bf16)
    be_arr = be_out[0]

    grid_spec = pltpu.PrefetchScalarGridSpec(
        num_scalar_prefetch=1,
        grid=(NBLK,),
        in_specs=[
            pl.BlockSpec((BLK, H), lambda bb, be: (bb, 0)),
            pl.BlockSpec((1, I, H), lambda bb, be: (be[bb], 0, 0)),
            pl.BlockSpec((1, I, H), lambda bb, be: (be[bb], 0, 0)),
            pl.BlockSpec((1, H, I), lambda bb, be: (be[bb], 0, 0)),
        ],
        out_specs=pl.BlockSpec((BLK, H), lambda bb, be: (bb, 0)),
    )
    y = pl.pallas_call(
        _ffn_sparse_kernel,
        grid_spec=grid_spec,
        out_shape=jax.ShapeDtypeStruct((NSLOT, H), jnp.float32),
    )(be_arr, disp, wg8, wu8, wd8)

    y2 = _s3_gather(s2_chunked, y).reshape(2, T, H)

    out = pl.pallas_call(
        _final_kernel,
        out_shape=jax.ShapeDtypeStruct((T, H), jnp.float32),
    )(x_bf, shared_gate_w.astype(bf16), shared_up_w.astype(bf16),
      shared_down_w.astype(bf16), y2, w2)

    return out.reshape(b, ss, h)


# speedup vs baseline: 19.3075x; 1.1527x over previous
import functools

import jax
import jax.numpy as jnp
from jax import lax
from jax.experimental import pallas as pl
from jax.experimental.pallas import tpu as pltpu
from jax.experimental.pallas import tpu_sc as plsc

H = 1024
HH = H // 2
I = 512
E = 8
NG = 4
T = 2048
BLK = 256
LOG_BLK = 8
NBLK = 24
NSLOT = NBLK * BLK
NTILES = 32
RCH = 128


def _pack_bf16(lo_f32, hi_f32):
    lo = lax.bitcast_convert_type(lo_f32.astype(jnp.bfloat16),
                                  jnp.uint16).astype(jnp.uint32)
    hi = lax.bitcast_convert_type(hi_f32.astype(jnp.bfloat16),
                                  jnp.uint16).astype(jnp.uint32)
    return lax.bitcast_convert_type(lo | (hi << 16), jnp.int32)


def _unpack_bf16(p):
    lo = lax.bitcast_convert_type(p << 16, jnp.float32)
    hi = lax.bitcast_convert_type(p & jnp.int32(-65536), jnp.float32)
    return lo, hi


def _routing_kernel(x_ref, gw_ref, eb_ref, w2_ref, s2_ref, be_ref, xp_ref):
    x = x_ref[...]
    gw = gw_ref[...]
    logits_t = lax.dot_general(gw, x, (((1,), (1,)), ((), ())),
                               preferred_element_type=jnp.float32)
    scores = jax.nn.sigmoid(logits_t)
    sfc = scores + eb_ref[...]
    s = [sfc[e:e + 1, :] for e in range(E)]
    sc = [scores[e:e + 1, :] for e in range(E)]
    g = [s[2 * i] + s[2 * i + 1] for i in range(NG)]
    gsel = []
    for i in range(NG):
        r = jnp.zeros_like(g[0])
        for j in range(NG):
            if j == i:
                continue
            beats = (g[j] > g[i]) | ((g[j] == g[i]) & (j < i))
            r = r + beats.astype(jnp.float32)
        gsel.append(r < 2.0)
    tmp = [jnp.where(gsel[e // 2], s[e], 0.0) for e in range(E)]
    esel = []
    for e in range(E):
        r = jnp.zeros_like(g[0])
        for e2 in range(E):
            if e2 == e:
                continue
            beats = (tmp[e2] > tmp[e]) | ((tmp[e2] == tmp[e]) & (e2 < e))
            r = r + beats.astype(jnp.float32)
        esel.append(r < 2.0)
    w = [jnp.where(esel[e], sc[e], 0.0) for e in range(E)]
    wsum = w[0]
    for e in range(1, E):
        wsum = wsum + w[e]
    inv = 1.0 / (wsum + 1e-20)
    wn = [w[e] * inv for e in range(E)]
    sel_f = [esel[e].astype(jnp.float32) for e in range(E)]
    sel_mat = jnp.concatenate(sel_f, axis=0)

    iot_r = lax.broadcasted_iota(jnp.int32, (T, T), 0)
    iot_c = lax.broadcasted_iota(jnp.int32, (T, T), 1)
    tril = (iot_r < iot_c).astype(jnp.float32)
    rank_f = lax.dot_general(sel_mat, tril, (((1,), (0,)), ((), ())),
                             preferred_element_type=jnp.float32)
    counts = lax.dot_general(sel_mat, jnp.ones((T, 1), jnp.float32),
                             (((1,), (0,)), ((), ())),
                             preferred_element_type=jnp.float32)
    c_i = counts.astype(jnp.int32)
    pc = ((c_i + BLK - 1) >> LOG_BLK) << LOG_BLK
    e_r = lax.broadcasted_iota(jnp.int32, (E, E), 0)
    e_c = lax.broadcasted_iota(jnp.int32, (E, E), 1)
    tril8 = (e_c < e_r).astype(jnp.float32)
    off = lax.dot_general(tril8, pc.astype(jnp.float32),
                          (((1,), (0,)), ((), ())),
                          preferred_element_type=jnp.float32
                          ).astype(jnp.int32)
    rank_i = rank_f.astype(jnp.int32)

    slot_f = (off + rank_i).astype(jnp.float32)
    cb = jnp.zeros((1, T), jnp.float32)
    slotA = jnp.zeros((1, T), jnp.float32)
    slotB = jnp.zeros((1, T), jnp.float32)
    wA = jnp.zeros((1, T), jnp.float32)
    wB = jnp.zeros((1, T), jnp.float32)
    for e in range(E):
        f = sel_f[e]
        first = f * (cb == 0.0).astype(jnp.float32)
        second = f * (cb == 1.0).astype(jnp.float32)
        row = slot_f[e:e + 1, :]
        slotA = slotA + first * row
        slotB = slotB + second * row
        wA = wA + first * wn[e]
        wB = wB + second * wn[e]
        cb = cb + f
    s2_ref[...] = jnp.concatenate([slotA, slotB], axis=0).astype(jnp.int32)
    zero = jnp.zeros((1, T), jnp.float32)
    w2_t = jnp.concatenate([wA, wB] + [zero] * (E - 2), axis=0)
    w2_ref[...] = w2_t.T

    bidx = lax.broadcasted_iota(jnp.int32, (E, 32), 1)
    boff = off >> LOG_BLK
    ge = (bidx >= boff).astype(jnp.int32)
    cnt = jnp.sum(ge, axis=0, keepdims=True)
    be_row = jnp.clip(cnt - 1, 0, E - 1)
    nb = jnp.sum(pc, axis=0, keepdims=True) >> LOG_BLK
    bi1 = lax.broadcasted_iota(jnp.int32, (1, 32), 1)
    be_row = jnp.where(bi1 == 31, nb, be_row)
    be_ref[...] = be_row + jnp.zeros((E, 32), jnp.int32)

    xp_ref[...] = _pack_bf16(x[:, :HH], x[:, HH:])


def _build_dispatch():
    mesh = plsc.VectorSubcoreMesh(core_axis_name="c", subcore_axis_name="s")
    rows_per = (2 * T) // NTILES

    @functools.partial(
        pl.kernel,
        mesh=mesh,
        out_type=jax.ShapeDtypeStruct((NSLOT, HH), jnp.int32),
        scratch_types=[
            pltpu.VMEM((1, RCH), jnp.int32),
            pltpu.VMEM((RCH, HH), jnp.int32),
            pltpu.SemaphoreType.DMA,
        ],
    )
    def _dispatch(s2_hbm, x_hbm, disp_hbm, idx_v, rows_v, sem):
        c = lax.axis_index("c")
        s = lax.axis_index("s")
        wid = s * 2 + c
        flat0 = pl.multiple_of(wid * rows_per, rows_per)
        t0 = pl.multiple_of(flat0 & (T - 1), rows_per)
        pltpu.sync_copy(s2_hbm.at[pl.ds(wid, 1)], idx_v)
        pltpu.sync_copy(x_hbm.at[pl.ds(t0, RCH)], rows_v)
        pltpu.async_copy(rows_v, disp_hbm.at[idx_v.at[0]], sem).wait()

    return _dispatch


def _build_gather():
    mesh = plsc.VectorSubcoreMesh(core_axis_name="c", subcore_axis_name="s")
    rows_per = (2 * T) // NTILES

    @functools.partial(
        pl.kernel,
        mesh=mesh,
        out_type=jax.ShapeDtypeStruct((2 * T, HH), jnp.int32),
        scratch_types=[
            pltpu.VMEM((1, RCH), jnp.int32),
            pltpu.VMEM((RCH, HH), jnp.int32),
            pltpu.SemaphoreType.DMA,
        ],
    )
    def _gather(idx_hbm, tab_hbm, out_hbm, idx_v, rows_v, sem):
        c = lax.axis_index("c")
        s = lax.axis_index("s")
        wid = s * 2 + c
        base = pl.multiple_of(wid * rows_per, rows_per)
        pltpu.sync_copy(idx_hbm.at[pl.ds(wid, 1)], idx_v)
        pltpu.async_copy(tab_hbm.at[idx_v.at[0]], rows_v, sem).wait()
        pltpu.sync_copy(rows_v, out_hbm.at[pl.ds(base, RCH)])

    return _gather


@functools.lru_cache(maxsize=None)
def _sc_kernels():
    return (_build_dispatch(), _build_gather())


def _sc_dispatch(s2_chunked, xp):
    return _sc_kernels()[0](s2_chunked, xp)


def _s3_gather(s2_chunked, yp):
    return _sc_kernels()[1](s2_chunked, yp)


def _ffn_sparse_kernel(be_ref, disp_ref, wg_ref, wu_ref, wd_ref, y_ref):
    b = pl.program_id(0)
    nb = be_ref[31]

    @pl.when(b < nb)
    def _():
        xlo_f, xhi_f = _unpack_bf16(disp_ref[...])
        xlo = xlo_f.astype(jnp.bfloat16)
        xhi = xhi_f.astype(jnp.bfloat16)
        wg = wg_ref[0]
        wu = wu_ref[0]
        hg = (lax.dot_general(xlo, wg[:, :HH], (((1,), (1,)), ((), ())),
                              preferred_element_type=jnp.float32) +
              lax.dot_general(xhi, wg[:, HH:], (((1,), (1,)), ((), ())),
                              preferred_element_type=jnp.float32))
        hu = (lax.dot_general(xlo, wu[:, :HH], (((1,), (1,)), ((), ())),
                              preferred_element_type=jnp.float32) +
              lax.dot_general(xhi, wu[:, HH:], (((1,), (1,)), ((), ())),
                              preferred_element_type=jnp.float32))
        inter = (hg * jax.nn.sigmoid(hg)) * hu
        y = lax.dot_general(inter.astype(jnp.bfloat16), wd_ref[0],
                            (((1,), (1,)), ((), ())),
                            preferred_element_type=jnp.float32)
        y_ref[...] = _pack_bf16(y[:, :HH], y[:, HH:])


def _final_kernel(x_ref, wsg_ref, wsu_ref, wsd_ref, y2_ref, w2_ref, out_ref):
    x = x_ref[...].astype(jnp.bfloat16)
    hg = lax.dot_general(x, wsg_ref[...], (((1,), (1,)), ((), ())),
                         preferred_element_type=jnp.float32)
    hu = lax.dot_general(x, wsu_ref[...], (((1,), (1,)), ((), ())),
                         preferred_element_type=jnp.float32)
    inter = (hg * jax.nn.sigmoid(hg)) * hu
    sh = lax.dot_general(inter.astype(jnp.bfloat16), wsd_ref[...],
                         (((1,), (1,)), ((), ())),
                         preferred_element_type=jnp.float32)
    loA, hiA = _unpack_bf16(y2_ref[0])
    loB, hiB = _unpack_bf16(y2_ref[1])
    wA = w2_ref[:, 0:1]
    wB = w2_ref[:, 1:2]
    out_ref[:, :HH] = sh[:, :HH] + loA * wA + loB * wB
    out_ref[:, HH:] = sh[:, HH:] + hiA * wA + hiB * wB


def kernel(hidden_states, gate_w, e_bias, gate_proj, up_proj, down_proj,
           shared_gate_w, shared_up_w, shared_down_w):
    b, ss, h = hidden_states.shape
    x = hidden_states.reshape(T, H)

    w2, s2, be_out, xp = pl.pallas_call(
        _routing_kernel,
        out_shape=(
            jax.ShapeDtypeStruct((T, E), jnp.float32),
            jax.ShapeDtypeStruct((2, T), jnp.int32),
            jax.ShapeDtypeStruct((E, 32), jnp.int32),
            jax.ShapeDtypeStruct((T, HH), jnp.int32),
        ),
    )(x, gate_w, e_bias.reshape(E, 1))

    s2_chunked = s2.reshape(2 * T // RCH, RCH)
    disp = _sc_dispatch(s2_chunked, xp)

    bf16 = jnp.bfloat16
    wg8 = gate_proj.astype(bf16)
    wu8 = up_proj.astype(bf16)
    wd8 = down_proj.astype(bf16)
    be_arr = be_out[0]

    grid_spec = pltpu.PrefetchScalarGridSpec(
        num_scalar_prefetch=1,
        grid=(NBLK,),
        in_specs=[
            pl.BlockSpec((BLK, HH), lambda bb, be: (bb, 0)),
            pl.BlockSpec((1, I, H), lambda bb, be: (be[bb], 0, 0)),
            pl.BlockSpec((1, I, H), lambda bb, be: (be[bb], 0, 0)),
            pl.BlockSpec((1, H, I), lambda bb, be: (be[bb], 0, 0)),
        ],
        out_specs=pl.BlockSpec((BLK, HH), lambda bb, be: (bb, 0)),
    )
    yp = pl.pallas_call(
        _ffn_sparse_kernel,
        grid_spec=grid_spec,
        out_shape=jax.ShapeDtypeStruct((NSLOT, HH), jnp.int32),
    )(be_arr, disp, wg8, wu8, wd8)

    y2 = _s3_gather(s2_chunked, yp).reshape(2, T, HH)

    out = pl.pallas_call(
        _final_kernel,
        out_shape=jax.ShapeDtypeStruct((T, H), jnp.float32),
    )(x, shared_gate_w.astype(bf16), shared_up_w.astype(bf16),
      shared_down_w.astype(bf16), y2, w2)

    return out.reshape(b, ss, h)


# speedup vs baseline: 22.1020x; 1.1447x over previous
import functools

import jax
import jax.numpy as jnp
from jax import lax
from jax.experimental import pallas as pl
from jax.experimental.pallas import tpu as pltpu
from jax.experimental.pallas import tpu_sc as plsc

H = 1024
HH = H // 2
I = 512
E = 8
NG = 4
T = 2048
BLK = 256
LOG_BLK = 8
NBLK = 24
NSLOT = NBLK * BLK
NTILES = 32
RCH = 128


def _pack_bf16(lo_f32, hi_f32):
    lo = lax.bitcast_convert_type(lo_f32.astype(jnp.bfloat16),
                                  jnp.uint16).astype(jnp.uint32)
    hi = lax.bitcast_convert_type(hi_f32.astype(jnp.bfloat16),
                                  jnp.uint16).astype(jnp.uint32)
    return lax.bitcast_convert_type(lo | (hi << 16), jnp.int32)


def _unpack_bf16(p):
    lo = lax.bitcast_convert_type(p << 16, jnp.float32)
    hi = lax.bitcast_convert_type(p & jnp.int32(-65536), jnp.float32)
    return lo, hi


def _routing_kernel(x_ref, gw_ref, eb_ref, w2_ref, s2_ref, be_ref, xp_ref):
    x = x_ref[...]
    gw = gw_ref[...]
    logits_t = lax.dot_general(gw, x, (((1,), (1,)), ((), ())),
                               preferred_element_type=jnp.float32)
    scores = jax.nn.sigmoid(logits_t)
    sfc = scores + eb_ref[...]
    s = [sfc[e:e + 1, :] for e in range(E)]
    sc = [scores[e:e + 1, :] for e in range(E)]
    g = [s[2 * i] + s[2 * i + 1] for i in range(NG)]
    gsel = []
    for i in range(NG):
        r = jnp.zeros_like(g[0])
        for j in range(NG):
            if j == i:
                continue
            beats = (g[j] > g[i]) | ((g[j] == g[i]) & (j < i))
            r = r + beats.astype(jnp.float32)
        gsel.append(r < 2.0)
    tmp = [jnp.where(gsel[e // 2], s[e], 0.0) for e in range(E)]
    esel = []
    for e in range(E):
        r = jnp.zeros_like(g[0])
        for e2 in range(E):
            if e2 == e:
                continue
            beats = (tmp[e2] > tmp[e]) | ((tmp[e2] == tmp[e]) & (e2 < e))
            r = r + beats.astype(jnp.float32)
        esel.append(r < 2.0)
    w = [jnp.where(esel[e], sc[e], 0.0) for e in range(E)]
    wsum = w[0]
    for e in range(1, E):
        wsum = wsum + w[e]
    inv = 1.0 / (wsum + 1e-20)
    wn = [w[e] * inv for e in range(E)]
    sel_f = [esel[e].astype(jnp.float32) for e in range(E)]
    sel_mat = jnp.concatenate(sel_f, axis=0)

    iot_r = lax.broadcasted_iota(jnp.int32, (T, T), 0)
    iot_c = lax.broadcasted_iota(jnp.int32, (T, T), 1)
    tril = (iot_r < iot_c).astype(jnp.float32)
    rank_f = lax.dot_general(sel_mat, tril, (((1,), (0,)), ((), ())),
                             preferred_element_type=jnp.float32)
    counts = lax.dot_general(sel_mat, jnp.ones((T, 1), jnp.float32),
                             (((1,), (0,)), ((), ())),
                             preferred_element_type=jnp.float32)
    c_i = counts.astype(jnp.int32)
    pc = ((c_i + BLK - 1) >> LOG_BLK) << LOG_BLK
    e_r = lax.broadcasted_iota(jnp.int32, (E, E), 0)
    e_c = lax.broadcasted_iota(jnp.int32, (E, E), 1)
    tril8 = (e_c < e_r).astype(jnp.float32)
    off = lax.dot_general(tril8, pc.astype(jnp.float32),
                          (((1,), (0,)), ((), ())),
                          preferred_element_type=jnp.float32
                          ).astype(jnp.int32)
    rank_i = rank_f.astype(jnp.int32)

    slot_f = (off + rank_i).astype(jnp.float32)
    cb = jnp.zeros((1, T), jnp.float32)
    slotA = jnp.zeros((1, T), jnp.float32)
    slotB = jnp.zeros((1, T), jnp.float32)
    wA = jnp.zeros((1, T), jnp.float32)
    wB = jnp.zeros((1, T), jnp.float32)
    for e in range(E):
        f = sel_f[e]
        first = f * (cb == 0.0).astype(jnp.float32)
        second = f * (cb == 1.0).astype(jnp.float32)
        row = slot_f[e:e + 1, :]
        slotA = slotA + first * row
        slotB = slotB + second * row
        wA = wA + first * wn[e]
        wB = wB + second * wn[e]
        cb = cb + f
    s2_ref[...] = jnp.concatenate([slotA, slotB], axis=0).astype(jnp.int32)
    zero = jnp.zeros((1, T), jnp.float32)
    w2_t = jnp.concatenate([wA, wB] + [zero] * (E - 2), axis=0)
    w2_ref[...] = w2_t.T

    bidx = lax.broadcasted_iota(jnp.int32, (E, 32), 1)
    boff = off >> LOG_BLK
    ge = (bidx >= boff).astype(jnp.int32)
    cnt = jnp.sum(ge, axis=0, keepdims=True)
    be_row = jnp.clip(cnt - 1, 0, E - 1)
    nb = jnp.sum(pc, axis=0, keepdims=True) >> LOG_BLK
    bi1 = lax.broadcasted_iota(jnp.int32, (1, 32), 1)
    be_row = jnp.where(bi1 == 31, nb, be_row)
    be_ref[...] = be_row + jnp.zeros((E, 32), jnp.int32)

    xp_ref[...] = _pack_bf16(x[:, :HH], x[:, HH:])


def _build_dispatch():
    mesh = plsc.VectorSubcoreMesh(core_axis_name="c", subcore_axis_name="s")
    rows_per = (2 * T) // NTILES

    @functools.partial(
        pl.kernel,
        mesh=mesh,
        out_type=jax.ShapeDtypeStruct((NSLOT, HH), jnp.int32),
        scratch_types=[
            pltpu.VMEM((1, RCH), jnp.int32),
            pltpu.VMEM((RCH, HH), jnp.int32),
            pltpu.SemaphoreType.DMA,
        ],
    )
    def _dispatch(s2_hbm, x_hbm, disp_hbm, idx_v, rows_v, sem):
        c = lax.axis_index("c")
        s = lax.axis_index("s")
        wid = s * 2 + c
        flat0 = pl.multiple_of(wid * rows_per, rows_per)
        t0 = pl.multiple_of(flat0 & (T - 1), rows_per)
        pltpu.sync_copy(s2_hbm.at[pl.ds(wid, 1)], idx_v)
        pltpu.sync_copy(x_hbm.at[pl.ds(t0, RCH)], rows_v)
        pltpu.async_copy(rows_v, disp_hbm.at[idx_v.at[0]], sem).wait()

    return _dispatch


def _build_gather():
    mesh = plsc.VectorSubcoreMesh(core_axis_name="c", subcore_axis_name="s")
    rows_per = (2 * T) // NTILES

    @functools.partial(
        pl.kernel,
        mesh=mesh,
        out_type=jax.ShapeDtypeStruct((2 * T, HH), jnp.int32),
        scratch_types=[
            pltpu.VMEM((1, RCH), jnp.int32),
            pltpu.VMEM((RCH, HH), jnp.int32),
            pltpu.SemaphoreType.DMA,
        ],
    )
    def _gather(idx_hbm, tab_hbm, out_hbm, idx_v, rows_v, sem):
        c = lax.axis_index("c")
        s = lax.axis_index("s")
        wid = s * 2 + c
        base = pl.multiple_of(wid * rows_per, rows_per)
        pltpu.sync_copy(idx_hbm.at[pl.ds(wid, 1)], idx_v)
        pltpu.async_copy(tab_hbm.at[idx_v.at[0]], rows_v, sem).wait()
        pltpu.sync_copy(rows_v, out_hbm.at[pl.ds(base, RCH)])

    return _gather


@functools.lru_cache(maxsize=None)
def _sc_kernels():
    return (_build_dispatch(), _build_gather())


def _sc_dispatch(s2_chunked, xp):
    return _sc_kernels()[0](s2_chunked, xp)


def _s3_gather(s2_chunked, yp):
    return _sc_kernels()[1](s2_chunked, yp)


def _ffn_sparse_kernel(be_ref, disp_ref, wg_ref, wu_ref, wd_ref, y_ref):
    b = pl.program_id(0)
    nb = be_ref[31]

    @pl.when(b < nb)
    def _():
        xlo_f, xhi_f = _unpack_bf16(disp_ref[...])
        xlo = xlo_f.astype(jnp.bfloat16)
        xhi = xhi_f.astype(jnp.bfloat16)
        wg = wg_ref[0]
        wu = wu_ref[0]
        hg = (lax.dot_general(xlo, wg[:, :HH].astype(jnp.bfloat16),
                              (((1,), (1,)), ((), ())),
                              preferred_element_type=jnp.float32) +
              lax.dot_general(xhi, wg[:, HH:].astype(jnp.bfloat16),
                              (((1,), (1,)), ((), ())),
                              preferred_element_type=jnp.float32))
        hu = (lax.dot_general(xlo, wu[:, :HH].astype(jnp.bfloat16),
                              (((1,), (1,)), ((), ())),
                              preferred_element_type=jnp.float32) +
              lax.dot_general(xhi, wu[:, HH:].astype(jnp.bfloat16),
                              (((1,), (1,)), ((), ())),
                              preferred_element_type=jnp.float32))
        inter = (hg * jax.nn.sigmoid(hg)) * hu
        y = lax.dot_general(inter.astype(jnp.bfloat16),
                            wd_ref[0].astype(jnp.bfloat16),
                            (((1,), (1,)), ((), ())),
                            preferred_element_type=jnp.float32)
        y_ref[...] = _pack_bf16(y[:, :HH], y[:, HH:])


def _final_kernel(x_ref, wsg_ref, wsu_ref, wsd_ref, y2_ref, w2_ref, out_ref):
    x = x_ref[...].astype(jnp.bfloat16)
    hg = lax.dot_general(x, wsg_ref[...].astype(jnp.bfloat16),
                         (((1,), (1,)), ((), ())),
                         preferred_element_type=jnp.float32)
    hu = lax.dot_general(x, wsu_ref[...].astype(jnp.bfloat16),
                         (((1,), (1,)), ((), ())),
                         preferred_element_type=jnp.float32)
    inter = (hg * jax.nn.sigmoid(hg)) * hu
    sh = lax.dot_general(inter.astype(jnp.bfloat16),
                         wsd_ref[...].astype(jnp.bfloat16),
                         (((1,), (1,)), ((), ())),
                         preferred_element_type=jnp.float32)
    loA, hiA = _unpack_bf16(y2_ref[0])
    loB, hiB = _unpack_bf16(y2_ref[1])
    wA = w2_ref[:, 0:1]
    wB = w2_ref[:, 1:2]
    out_ref[:, :HH] = sh[:, :HH] + loA * wA + loB * wB
    out_ref[:, HH:] = sh[:, HH:] + hiA * wA + hiB * wB


def kernel(hidden_states, gate_w, e_bias, gate_proj, up_proj, down_proj,
           shared_gate_w, shared_up_w, shared_down_w):
    b, ss, h = hidden_states.shape
    x = hidden_states.reshape(T, H)

    w2, s2, be_out, xp = pl.pallas_call(
        _routing_kernel,
        out_shape=(
            jax.ShapeDtypeStruct((T, E), jnp.float32),
            jax.ShapeDtypeStruct((2, T), jnp.int32),
            jax.ShapeDtypeStruct((E, 32), jnp.int32),
            jax.ShapeDtypeStruct((T, HH), jnp.int32),
        ),
    )(x, gate_w, e_bias.reshape(E, 1))

    s2_chunked = s2.reshape(2 * T // RCH, RCH)
    disp = _sc_dispatch(s2_chunked, xp)

    be_arr = be_out[0]

    grid_spec = pltpu.PrefetchScalarGridSpec(
        num_scalar_prefetch=1,
        grid=(NBLK,),
        in_specs=[
            pl.BlockSpec((BLK, HH), lambda bb, be: (bb, 0)),
            pl.BlockSpec((1, I, H), lambda bb, be: (be[bb], 0, 0)),
            pl.BlockSpec((1, I, H), lambda bb, be: (be[bb], 0, 0)),
            pl.BlockSpec((1, H, I), lambda bb, be: (be[bb], 0, 0)),
        ],
        out_specs=pl.BlockSpec((BLK, HH), lambda bb, be: (bb, 0)),
    )
    yp = pl.pallas_call(
        _ffn_sparse_kernel,
        grid_spec=grid_spec,
        out_shape=jax.ShapeDtypeStruct((NSLOT, HH), jnp.int32),
    )(be_arr, disp, gate_proj, up_proj, down_proj)

    y2 = _s3_gather(s2_chunked, yp).reshape(2, T, HH)

    out = pl.pallas_call(
        _final_kernel,
        out_shape=jax.ShapeDtypeStruct((T, H), jnp.float32),
    )(x, shared_gate_w, shared_up_w, shared_down_w, y2, w2)

    return out.reshape(b, ss, h)
